# Initial kernel scaffold; baseline (speedup 1.0000x reference)
import jax
import jax.numpy as jnp
from jax.experimental import pallas as pl


def kernel(x, emb_table, lin_w, lin_b):
    raise NotImplementedError("write your pallas kernel here")


# speedup vs baseline: 1.1532x; 1.1532x over previous
import functools

import jax
import jax.numpy as jnp
from jax import lax
from jax.experimental import pallas as pl
from jax.experimental.pallas import tpu as pltpu
from jax.experimental.pallas import tpu_sc as plsc

_NC = 2
_NS = 16
_NW = _NC * _NS

_LANES = 16


def _make_pool(B, S, E, V):
    b_per_w = B // _NW
    n_chunks = 5
    chunk = S // n_chunks
    assert chunk * n_chunks == S and chunk <= 128 and chunk % 8 == 0
    groups = E // _LANES

    mesh = plsc.VectorSubcoreMesh(core_axis_name="c", subcore_axis_name="s")

    @functools.partial(
        pl.kernel,
        mesh=mesh,
        out_type=jax.ShapeDtypeStruct((B * E,), jnp.float32),
        scratch_types=[
            pltpu.VMEM((b_per_w * S,), jnp.int32),
            pltpu.VMEM((chunk, E), jnp.float32),
            pltpu.VMEM((b_per_w * E,), jnp.float32),
            pltpu.SemaphoreType.DMA,
        ],
        compiler_params=pltpu.CompilerParams(use_tc_tiling_on_sc=False),
    )
    def pool(x_hbm, table_hbm, out_hbm, idx_v, rows_v, acc_v, sem):
        wid = lax.axis_index("s") * _NC + lax.axis_index("c")
        base = wid * b_per_w
        pltpu.sync_copy(
            x_hbm.at[pl.ds(pl.multiple_of(base * S, 8), b_per_w * S)], idx_v
        )

        def row_body(i, carry):
            accs = [jnp.zeros((_LANES,), jnp.float32) for _ in range(groups)]
            for c in range(n_chunks):
                off = pl.multiple_of(i * S + c * chunk, 8)
                pltpu.async_copy(
                    table_hbm.at[idx_v.at[pl.ds(off, chunk)]],
                    rows_v,
                    sem,
                ).wait()
                for j in range(chunk):
                    for g in range(groups):
                        accs[g] = accs[g] + rows_v[j, pl.ds(g * _LANES, _LANES)]
            scale = jnp.float32(1.0 / S)
            for g in range(groups):
                aoff = pl.multiple_of(i * E + g * _LANES, 8)
                acc_v[pl.ds(aoff, _LANES)] = accs[g] * scale
            return carry

        lax.fori_loop(0, b_per_w, row_body, 0)
        pltpu.sync_copy(
            acc_v, out_hbm.at[pl.ds(pl.multiple_of(base * E, 8), b_per_w * E)]
        )

    return pool


def _mm_body(p_ref, w_ref, b_ref, o_ref):
    o_ref[...] = (
        lax.dot_general(
            p_ref[...],
            w_ref[...],
            (((1,), (1,)), ((), ())),
            preferred_element_type=jnp.float32,
        )
        + b_ref[...]
    )


def _matmul(pooled, lin_w, lin_b2d, v_blk=2048):
    B, E = pooled.shape
    V = lin_w.shape[0]
    nb = pl.cdiv(V, v_blk)
    return pl.pallas_call(
        _mm_body,
        grid=(nb,),
        in_specs=[
            pl.BlockSpec((B, E), lambda i: (0, 0)),
            pl.BlockSpec((v_blk, E), lambda i: (i, 0)),
            pl.BlockSpec((1, v_blk), lambda i: (0, i)),
        ],
        out_specs=pl.BlockSpec((B, v_blk), lambda i: (0, i)),
        out_shape=jax.ShapeDtypeStruct((B, V), jnp.float32),
        compiler_params=pltpu.CompilerParams(
            dimension_semantics=("arbitrary",),
        ),
    )(pooled, lin_w, lin_b2d)


@jax.jit
def kernel(x, emb_table, lin_w, lin_b):
    B, S = x.shape
    V, E = emb_table.shape
    pooled = _make_pool(B, S, E, V)(x.reshape(B * S), emb_table)
    pooled = pooled.reshape(B, E)
    return _matmul(pooled, lin_w, lin_b.reshape(1, V))


# speedup vs baseline: 1.2983x; 1.1258x over previous
import functools

import jax
import jax.numpy as jnp
from jax import lax
from jax.experimental import pallas as pl
from jax.experimental.pallas import tpu as pltpu
from jax.experimental.pallas import tpu_sc as plsc

_NC = 2
_NS = 16
_NW = _NC * _NS

_LANES = 16


def _make_pool(B, S, E, V):
    b_per_w = B // _NW
    n_chunks = 5
    chunk = S // n_chunks
    assert chunk * n_chunks == S and chunk <= 128 and chunk % 8 == 0
    groups = E // _LANES

    mesh = plsc.VectorSubcoreMesh(core_axis_name="c", subcore_axis_name="s")

    @functools.partial(
        pl.kernel,
        mesh=mesh,
        out_type=jax.ShapeDtypeStruct((B * E,), jnp.float32),
        scratch_types=[
            pltpu.VMEM((b_per_w * S,), jnp.int32),
            pltpu.VMEM((n_chunks, chunk, E), jnp.float32),
            pltpu.VMEM((n_chunks, chunk, E), jnp.float32),
            pltpu.VMEM((b_per_w * E,), jnp.float32),
            pltpu.SemaphoreType.DMA,
            pltpu.SemaphoreType.DMA,
        ],
        compiler_params=pltpu.CompilerParams(use_tc_tiling_on_sc=False),
    )
    def pool(x_hbm, table_hbm, out_hbm, idx_v, buf_a, buf_b, acc_v, sem_a, sem_b):
        wid = lax.axis_index("s") * _NC + lax.axis_index("c")
        base = wid * b_per_w
        pltpu.sync_copy(
            x_hbm.at[pl.ds(pl.multiple_of(base * S, 8), b_per_w * S)], idx_v
        )

        def src(i, c):
            off = pl.multiple_of(i * S + c * chunk, 8)
            return table_hbm.at[idx_v.at[pl.ds(off, chunk)]]

        def fire(i, buf, sem):
            for c in range(n_chunks):
                pltpu.async_copy(src(i, c), buf.at[c], sem)

        def reduce_store(i, buf, sem):
            for c in range(n_chunks):
                pltpu.make_async_copy(src(i, c), buf.at[c], sem).wait()
            accs = [jnp.zeros((_LANES,), jnp.float32) for _ in range(groups)]
            for c in range(n_chunks):
                for j in range(chunk):
                    for g in range(groups):
                        accs[g] = accs[g] + buf[c, j, pl.ds(g * _LANES, _LANES)]
            scale = jnp.float32(1.0 / S)
            for g in range(groups):
                aoff = pl.multiple_of(i * E + g * _LANES, 8)
                acc_v[pl.ds(aoff, _LANES)] = accs[g] * scale

        fire(0, buf_a, sem_a)

        def body(k, carry):
            fire(2 * k + 1, buf_b, sem_b)
            reduce_store(2 * k, buf_a, sem_a)

            @pl.when(k < b_per_w // 2 - 1)
            def _():
                fire(2 * k + 2, buf_a, sem_a)

            reduce_store(2 * k + 1, buf_b, sem_b)
            return carry

        lax.fori_loop(0, b_per_w // 2, body, 0)
        pltpu.sync_copy(
            acc_v, out_hbm.at[pl.ds(pl.multiple_of(base * E, 8), b_per_w * E)]
        )

    return pool


def _mm_body(p_ref, w_ref, b_ref, o_ref):
    o_ref[...] = (
        lax.dot_general(
            p_ref[...],
            w_ref[...],
            (((1,), (1,)), ((), ())),
            preferred_element_type=jnp.float32,
        )
        + b_ref[...]
    )


def _matmul(pooled, lin_w, lin_b2d, v_blk=2048):
    B, E = pooled.shape
    V = lin_w.shape[0]
    nb = pl.cdiv(V, v_blk)
    return pl.pallas_call(
        _mm_body,
        grid=(nb,),
        in_specs=[
            pl.BlockSpec((B, E), lambda i: (0, 0)),
            pl.BlockSpec((v_blk, E), lambda i: (i, 0)),
            pl.BlockSpec((1, v_blk), lambda i: (0, i)),
        ],
        out_specs=pl.BlockSpec((B, v_blk), lambda i: (0, i)),
        out_shape=jax.ShapeDtypeStruct((B, V), jnp.float32),
        compiler_params=pltpu.CompilerParams(
            dimension_semantics=("parallel",),
        ),
    )(pooled, lin_w, lin_b2d)


@jax.jit
def kernel(x, emb_table, lin_w, lin_b):
    B, S = x.shape
    V, E = emb_table.shape
    pooled = _make_pool(B, S, E, V)(x.reshape(B * S), emb_table)
    pooled = pooled.reshape(B, E)
    return _matmul(pooled, lin_w, lin_b.reshape(1, V))
